# Initial kernel scaffold; baseline (speedup 1.0000x reference)
import jax
import jax.numpy as jnp
from jax.experimental import pallas as pl


def kernel(pos, pos_embd):
    raise NotImplementedError("write your pallas kernel here")


# speedup vs baseline: 1.6547x; 1.6547x over previous
import functools

import jax
import jax.numpy as jnp
from jax import lax
from jax.experimental import pallas as pl
from jax.experimental.pallas import tpu as pltpu
from jax.experimental.pallas import tpu_sc as plsc

D = 768
B = 16384
NC = 2
NS = 16
NW = NC * NS
B_PER_W = B // NW
CHUNK = 64
NCHUNK = B_PER_W // CHUNK
NBUF = 2


def _gather_body(table_hbm, idx_hbm, out_hbm, idx_v, rows_v, sem0, sem1):
    sems = (sem0, sem1)
    wid = lax.axis_index("s") * NC + lax.axis_index("c")
    base = wid * B_PER_W
    pltpu.sync_copy(idx_hbm.at[pl.ds(base, B_PER_W)], idx_v)
    copies = [None] * NCHUNK
    for i in range(NBUF):
        copies[i] = pltpu.async_copy(
            table_hbm.at[idx_v.at[pl.ds(i * CHUNK, CHUNK)]],
            rows_v.at[i % NBUF],
            sems[i % NBUF],
        )
    for i in range(NCHUNK):
        copies[i].wait()
        pltpu.sync_copy(rows_v.at[i % NBUF], out_hbm.at[pl.ds(base + i * CHUNK, CHUNK)])
        nxt = i + NBUF
        if nxt < NCHUNK:
            copies[nxt] = pltpu.async_copy(
                table_hbm.at[idx_v.at[pl.ds(nxt * CHUNK, CHUNK)]],
                rows_v.at[nxt % NBUF],
                sems[nxt % NBUF],
            )


@jax.jit
def _gather(pos, pos_embd):
    mesh = plsc.VectorSubcoreMesh(core_axis_name="c", subcore_axis_name="s")
    run = pl.kernel(
        _gather_body,
        mesh=mesh,
        out_type=jax.ShapeDtypeStruct((B, D), jnp.float32),
        scratch_types=[
            pltpu.VMEM((B_PER_W,), jnp.int32),
            pltpu.VMEM((NBUF, CHUNK, D), jnp.float32),
            pltpu.SemaphoreType.DMA,
            pltpu.SemaphoreType.DMA,
        ],
    )
    return run(pos_embd, pos)


def kernel(pos, pos_embd):
    return _gather(pos, pos_embd)
